# baseline (speedup 1.0000x reference)
import functools

import jax
import jax.numpy as jnp
from jax.experimental import pallas as pl
from jax.experimental.pallas import tpu as pltpu

_H = 128
_W = 2 * _H
_NB = 8192


def _body(steps, x_ref, tw1_ref, pw1_ref, tb1_ref, pb1_ref,
          tw2_ref, pw2_ref, tb2_ref, pb2_ref, w3_ref, b3_ref,
          mean_ref, m2_ref, count_ref,
          out_ref, racc_ref, acc1_ref, acc2_ref):
    i = pl.program_id(0)
    x = x_ref[...]
    reps = _NB // 128
    tb1 = pltpu.repeat(tb1_ref[...], reps, axis=1)
    pb1 = pltpu.repeat(pb1_ref[...], reps, axis=1)
    tb2 = pltpu.repeat(tb2_ref[...], reps, axis=1)
    pb2 = pltpu.repeat(pb2_ref[...], reps, axis=1)
    b3 = pltpu.repeat(b3_ref[...], reps, axis=1)
    ht = jnp.maximum(jnp.dot(tw1_ref[...], x, preferred_element_type=jnp.float32) + tb1, 0.0)
    hp = jnp.maximum(jnp.dot(pw1_ref[...], x, preferred_element_type=jnp.float32) + pb1, 0.0)
    ht = jnp.maximum(jnp.dot(tw2_ref[...], ht, preferred_element_type=jnp.float32) + tb2, 0.0)
    hp = jnp.maximum(jnp.dot(pw2_ref[...], hp, preferred_element_type=jnp.float32) + pb2, 0.0)
    h = jnp.concatenate([ht, hp], axis=0)
    d = jnp.dot(w3_ref[...], h, preferred_element_type=jnp.float32) + b3
    s16 = (d * d).reshape(16, 8, _NB)
    s8 = jnp.sum(s16, axis=0)
    r_row = jnp.sum(s8.reshape(1, 8, _NB), axis=1) * (1.0 / _H)
    racc_ref[pl.ds(i, 1)] = r_row[:, None, :]

    @pl.when(i == 0)
    def _init():
        acc1_ref[...] = jnp.zeros_like(acc1_ref)
        acc2_ref[...] = jnp.zeros_like(acc2_ref)

    acc1_ref[...] += r_row
    acc2_ref[...] += r_row * r_row

    @pl.when(i == steps - 1)
    def _final():
        n = jnp.float32(steps * _NB)
        s1 = jnp.sum(acc1_ref[...])
        s2 = jnp.sum(acc2_ref[...])
        bm = s1 / n
        bm2 = s2 - n * bm * bm
        cnt = count_ref[0]
        new_count = cnt + n
        delta = bm - mean_ref[0]
        new_mean = mean_ref[0] + delta * n / new_count
        new_m2 = m2_ref[0] + bm2 + delta * delta * cnt * n / new_count
        std = jnp.where(new_count > 1.0, jnp.sqrt(new_m2 / (new_count - 1.0)), 1.0)
        inv = 1.0 / (std + 1e-8)
        r_all = racc_ref[...].reshape(steps, _NB)
        out_ref[...] = (r_all - new_mean) * inv


def kernel(obs, reward_mean, reward_m2, reward_count,
           tW1, tb1, tW2, tb2, tW3, tb3,
           pW1, pb1, pW2, pb2, pW3, pb3):
    batch, obs_dim = obs.shape
    w3 = jnp.concatenate([tW3, -pW3], axis=1)
    b1t = jnp.broadcast_to(tb1[:, None], (_H, 128))
    b1p = jnp.broadcast_to(pb1[:, None], (_H, 128))
    b2t = jnp.broadcast_to(tb2[:, None], (_H, 128))
    b2p = jnp.broadcast_to(pb2[:, None], (_H, 128))
    b3 = jnp.broadcast_to((tb3 - pb3)[:, None], (_H, 128))

    xt = obs.T
    steps = batch // _NB

    normalized = pl.pallas_call(
        functools.partial(_body, steps),
        grid=(steps,),
        in_specs=[
            pl.BlockSpec((obs_dim, _NB), lambda i: (0, i)),
            pl.BlockSpec((_H, obs_dim), lambda i: (0, 0)),
            pl.BlockSpec((_H, obs_dim), lambda i: (0, 0)),
            pl.BlockSpec((_H, 128), lambda i: (0, 0)),
            pl.BlockSpec((_H, 128), lambda i: (0, 0)),
            pl.BlockSpec((_H, _H), lambda i: (0, 0)),
            pl.BlockSpec((_H, _H), lambda i: (0, 0)),
            pl.BlockSpec((_H, 128), lambda i: (0, 0)),
            pl.BlockSpec((_H, 128), lambda i: (0, 0)),
            pl.BlockSpec((_H, _W), lambda i: (0, 0)),
            pl.BlockSpec((_H, 128), lambda i: (0, 0)),
            pl.BlockSpec(memory_space=pltpu.SMEM),
            pl.BlockSpec(memory_space=pltpu.SMEM),
            pl.BlockSpec(memory_space=pltpu.SMEM),
        ],
        out_specs=pl.BlockSpec((steps, _NB), lambda i: (0, 0)),
        out_shape=jax.ShapeDtypeStruct((steps, _NB), jnp.float32),
        scratch_shapes=[
            pltpu.VMEM((steps, 1, _NB), jnp.float32),
            pltpu.VMEM((1, _NB), jnp.float32),
            pltpu.VMEM((1, _NB), jnp.float32),
        ],
        compiler_params=pltpu.CompilerParams(
            dimension_semantics=("arbitrary",),
        ),
    )(xt, tW1, pW1, b1t, b1p, tW2, pW2, b2t, b2p, w3, b3,
      reward_mean, reward_m2, reward_count)

    return normalized.reshape(batch)


# speedup vs baseline: 1.0474x; 1.0474x over previous
import functools

import jax
import jax.numpy as jnp
from jax.experimental import pallas as pl
from jax.experimental.pallas import tpu as pltpu

_H = 128
_W = 2 * _H
_NB = 16384


def _body(steps, x_ref, w1_ref, b1_ref, w2_ref, b2_ref, w3_ref, b3_ref,
          mean_ref, m2_ref, count_ref,
          out_ref, racc_ref, acc1_ref, acc2_ref):
    i = pl.program_id(0)
    x = x_ref[...]
    reps = _NB // 128
    b1 = pltpu.repeat(b1_ref[...], reps, axis=1)
    b2 = pltpu.repeat(b2_ref[...], reps, axis=1)
    b3 = pltpu.repeat(b3_ref[...], reps, axis=1)
    h = jnp.dot(w1_ref[...], x, preferred_element_type=jnp.float32)
    h = jnp.maximum(h + b1, 0.0)
    h = jnp.dot(w2_ref[...], h, preferred_element_type=jnp.float32)
    h = jnp.maximum(h + b2, 0.0)
    d = jnp.dot(w3_ref[...], h, preferred_element_type=jnp.float32) + b3
    s16 = (d * d).reshape(16, 8, _NB)
    s8 = jnp.sum(s16, axis=0)
    r_row = jnp.sum(s8.reshape(1, 8, _NB), axis=1) * (1.0 / _H)
    racc_ref[pl.ds(i, 1)] = r_row[:, None, :]

    @pl.when(i == 0)
    def _init():
        acc1_ref[...] = jnp.zeros_like(acc1_ref)
        acc2_ref[...] = jnp.zeros_like(acc2_ref)

    acc1_ref[...] += r_row
    acc2_ref[...] += r_row * r_row

    @pl.when(i == steps - 1)
    def _final():
        n = jnp.float32(steps * _NB)
        s1 = jnp.sum(acc1_ref[...])
        s2 = jnp.sum(acc2_ref[...])
        bm = s1 / n
        bm2 = s2 - n * bm * bm
        cnt = count_ref[0]
        new_count = cnt + n
        delta = bm - mean_ref[0]
        new_mean = mean_ref[0] + delta * n / new_count
        new_m2 = m2_ref[0] + bm2 + delta * delta * cnt * n / new_count
        std = jnp.where(new_count > 1.0, jnp.sqrt(new_m2 / (new_count - 1.0)), 1.0)
        inv = 1.0 / (std + 1e-8)
        r_all = racc_ref[...].reshape(steps, _NB)
        out_ref[...] = (r_all - new_mean) * inv


def kernel(obs, reward_mean, reward_m2, reward_count,
           tW1, tb1, tW2, tb2, tW3, tb3,
           pW1, pb1, pW2, pb2, pW3, pb3):
    batch, obs_dim = obs.shape
    z = jnp.zeros((_H, _H), jnp.float32)
    w1 = jnp.concatenate([tW1, pW1], axis=0)
    w2 = jnp.concatenate(
        [jnp.concatenate([tW2, z], axis=1),
         jnp.concatenate([z, pW2], axis=1)], axis=0)
    w3 = jnp.concatenate([tW3, -pW3], axis=1)
    b1 = jnp.broadcast_to(jnp.concatenate([tb1, pb1])[:, None], (_W, 128))
    b2 = jnp.broadcast_to(jnp.concatenate([tb2, pb2])[:, None], (_W, 128))
    b3 = jnp.broadcast_to((tb3 - pb3)[:, None], (_H, 128))

    xt = obs.T
    steps = batch // _NB

    normalized = pl.pallas_call(
        functools.partial(_body, steps),
        grid=(steps,),
        in_specs=[
            pl.BlockSpec((obs_dim, _NB), lambda i: (0, i)),
            pl.BlockSpec((_W, obs_dim), lambda i: (0, 0)),
            pl.BlockSpec((_W, 128), lambda i: (0, 0)),
            pl.BlockSpec((_W, _W), lambda i: (0, 0)),
            pl.BlockSpec((_W, 128), lambda i: (0, 0)),
            pl.BlockSpec((_H, _W), lambda i: (0, 0)),
            pl.BlockSpec((_H, 128), lambda i: (0, 0)),
            pl.BlockSpec(memory_space=pltpu.SMEM),
            pl.BlockSpec(memory_space=pltpu.SMEM),
            pl.BlockSpec(memory_space=pltpu.SMEM),
        ],
        out_specs=pl.BlockSpec((steps, _NB), lambda i: (0, 0)),
        out_shape=jax.ShapeDtypeStruct((steps, _NB), jnp.float32),
        scratch_shapes=[
            pltpu.VMEM((steps, 1, _NB), jnp.float32),
            pltpu.VMEM((1, _NB), jnp.float32),
            pltpu.VMEM((1, _NB), jnp.float32),
        ],
        compiler_params=pltpu.CompilerParams(
            dimension_semantics=("arbitrary",),
        ),
    )(xt, w1, b1, w2, b2, w3, b3, reward_mean, reward_m2, reward_count)

    return normalized.reshape(batch)


# speedup vs baseline: 1.0577x; 1.0098x over previous
import functools

import jax
import jax.numpy as jnp
from jax.experimental import pallas as pl
from jax.experimental.pallas import tpu as pltpu

_H = 128
_W = 2 * _H
_NB = 32768


def _body(steps, x_ref, w1_ref, b1_ref, w2_ref, b2_ref, w3_ref, b3_ref,
          mean_ref, m2_ref, count_ref,
          out_ref, racc_ref, acc1_ref, acc2_ref):
    i = pl.program_id(0)
    x = x_ref[...]
    reps = _NB // 128
    b1 = pltpu.repeat(b1_ref[...], reps, axis=1)
    b2 = pltpu.repeat(b2_ref[...], reps, axis=1)
    b3 = pltpu.repeat(b3_ref[...], reps, axis=1)
    h = jnp.dot(w1_ref[...], x, preferred_element_type=jnp.float32)
    h = jnp.maximum(h + b1, 0.0)
    h = jnp.dot(w2_ref[...], h, preferred_element_type=jnp.float32)
    h = jnp.maximum(h + b2, 0.0)
    d = jnp.dot(w3_ref[...], h, preferred_element_type=jnp.float32) + b3
    s16 = (d * d).reshape(16, 8, _NB)
    s8 = jnp.sum(s16, axis=0)
    r_row = jnp.sum(s8.reshape(1, 8, _NB), axis=1) * (1.0 / _H)
    racc_ref[pl.ds(i, 1)] = r_row[:, None, :]

    @pl.when(i == 0)
    def _init():
        acc1_ref[...] = jnp.zeros_like(acc1_ref)
        acc2_ref[...] = jnp.zeros_like(acc2_ref)

    acc1_ref[...] += r_row
    acc2_ref[...] += r_row * r_row

    @pl.when(i == steps - 1)
    def _final():
        n = jnp.float32(steps * _NB)
        s1 = jnp.sum(acc1_ref[...])
        s2 = jnp.sum(acc2_ref[...])
        bm = s1 / n
        bm2 = s2 - n * bm * bm
        cnt = count_ref[0]
        new_count = cnt + n
        delta = bm - mean_ref[0]
        new_mean = mean_ref[0] + delta * n / new_count
        new_m2 = m2_ref[0] + bm2 + delta * delta * cnt * n / new_count
        std = jnp.where(new_count > 1.0, jnp.sqrt(new_m2 / (new_count - 1.0)), 1.0)
        inv = 1.0 / (std + 1e-8)
        r_all = racc_ref[...].reshape(steps, _NB)
        out_ref[...] = (r_all - new_mean) * inv


def kernel(obs, reward_mean, reward_m2, reward_count,
           tW1, tb1, tW2, tb2, tW3, tb3,
           pW1, pb1, pW2, pb2, pW3, pb3):
    batch, obs_dim = obs.shape
    z = jnp.zeros((_H, _H), jnp.float32)
    w1 = jnp.concatenate([tW1, pW1], axis=0)
    w2 = jnp.concatenate(
        [jnp.concatenate([tW2, z], axis=1),
         jnp.concatenate([z, pW2], axis=1)], axis=0)
    w3 = jnp.concatenate([tW3, -pW3], axis=1)
    b1 = jnp.broadcast_to(jnp.concatenate([tb1, pb1])[:, None], (_W, 128))
    b2 = jnp.broadcast_to(jnp.concatenate([tb2, pb2])[:, None], (_W, 128))
    b3 = jnp.broadcast_to((tb3 - pb3)[:, None], (_H, 128))

    xt = obs.T
    steps = batch // _NB

    normalized = pl.pallas_call(
        functools.partial(_body, steps),
        grid=(steps,),
        in_specs=[
            pl.BlockSpec((obs_dim, _NB), lambda i: (0, i)),
            pl.BlockSpec((_W, obs_dim), lambda i: (0, 0)),
            pl.BlockSpec((_W, 128), lambda i: (0, 0)),
            pl.BlockSpec((_W, _W), lambda i: (0, 0)),
            pl.BlockSpec((_W, 128), lambda i: (0, 0)),
            pl.BlockSpec((_H, _W), lambda i: (0, 0)),
            pl.BlockSpec((_H, 128), lambda i: (0, 0)),
            pl.BlockSpec(memory_space=pltpu.SMEM),
            pl.BlockSpec(memory_space=pltpu.SMEM),
            pl.BlockSpec(memory_space=pltpu.SMEM),
        ],
        out_specs=pl.BlockSpec((steps, _NB), lambda i: (0, 0)),
        out_shape=jax.ShapeDtypeStruct((steps, _NB), jnp.float32),
        scratch_shapes=[
            pltpu.VMEM((steps, 1, _NB), jnp.float32),
            pltpu.VMEM((1, _NB), jnp.float32),
            pltpu.VMEM((1, _NB), jnp.float32),
        ],
        compiler_params=pltpu.CompilerParams(
            dimension_semantics=("arbitrary",),
        ),
    )(xt, w1, b1, w2, b2, w3, b3, reward_mean, reward_m2, reward_count)

    return normalized.reshape(batch)
